# baseline (speedup 1.0000x reference)
import functools

import jax
import jax.numpy as jnp
from jax import lax
from jax.experimental import pallas as pl
from jax.experimental.pallas import tpu as pltpu
from jax.experimental.pallas import tpu_sc as plsc

ESIZE = 64
EH = 32
NBUF = 8

_info = plsc.get_sparse_core_info()
NC, NS = _info.num_cores, _info.num_subcores
NW = NC * NS


@jax.jit
def _gather_rows(x, emb_a, emb_b):
    nb, nseq = x.shape
    assert nseq <= 128
    xrows_per_w = nb // NW
    ngrp = xrows_per_w // NBUF
    assert ngrp * NBUF * NW == nb
    mesh = plsc.VectorSubcoreMesh(core_axis_name="c", subcore_axis_name="s")

    @functools.partial(
        pl.kernel,
        out_type=jax.ShapeDtypeStruct((nb, nseq, ESIZE), jnp.float32),
        mesh=mesh,
        scratch_types=[
            pltpu.VMEM((xrows_per_w, nseq), jnp.int32),
            pltpu.VMEM((NBUF, nseq, EH), jnp.float32),
            pltpu.VMEM((NBUF, nseq, EH), jnp.float32),
            pltpu.SemaphoreType.DMA((NBUF,)),
            pltpu.SemaphoreType.DMA((NBUF,)),
        ],
        compiler_params=pltpu.CompilerParams(use_tc_tiling_on_sc=False),
    )
    def k(a_hbm, b_hbm, x_hbm, out_hbm, idx_v, rows_a, rows_b, gsem, ssem):
        wid = lax.axis_index("s") * NC + lax.axis_index("c")
        base = wid * xrows_per_w
        pltpu.sync_copy(x_hbm.at[pl.ds(base, xrows_per_w)], idx_v)

        def gathers(r, b):
            idx = idx_v.at[r]
            return (
                pltpu.make_async_copy(a_hbm.at[idx], rows_a.at[b], gsem.at[b]),
                pltpu.make_async_copy(b_hbm.at[idx], rows_b.at[b], gsem.at[b]),
            )

        def stores(r, b):
            row = out_hbm.at[base + r]
            return (
                pltpu.make_async_copy(rows_a.at[b], row.at[:, pl.ds(0, EH)], ssem.at[b]),
                pltpu.make_async_copy(rows_b.at[b], row.at[:, pl.ds(EH, EH)], ssem.at[b]),
            )

        def start(descs):
            for d in descs:
                d.start()

        def wait(descs):
            for d in descs:
                d.wait()

        for b in range(NBUF):
            start(gathers(b, b))

        def group(g, _):
            r0 = g * NBUF
            for b in range(NBUF):
                r = r0 + b
                wait(gathers(r, b))
                start(stores(r, b))
                wait(stores(r, b))
                start(gathers(r + NBUF, b))
            return _

        lax.fori_loop(0, ngrp - 1, group, None)

        r0 = (ngrp - 1) * NBUF
        for b in range(NBUF):
            r = r0 + b
            wait(gathers(r, b))
            start(stores(r, b))
            wait(stores(r, b))

    return k(emb_a, emb_b, x)


def kernel(x, emb):
    return _gather_rows(x.astype(jnp.int32), emb[:, :EH], emb[:, EH:])


# speedup vs baseline: 1.6153x; 1.6153x over previous
import functools

import jax
import jax.numpy as jnp
from jax import lax
from jax.experimental import pallas as pl
from jax.experimental.pallas import tpu as pltpu
from jax.experimental.pallas import tpu_sc as plsc

ESIZE = 64
NBUF = 8

_info = plsc.get_sparse_core_info()
NC, NS = _info.num_cores, _info.num_subcores
NW = NC * NS


@jax.jit
def _gather_rows(x_eo, emb):
    nb, _, nsh = x_eo.shape
    xrows_per_w = nb // NW
    ngrp = xrows_per_w // NBUF
    assert ngrp * NBUF * NW == nb
    mesh = plsc.VectorSubcoreMesh(core_axis_name="c", subcore_axis_name="s")

    @functools.partial(
        pl.kernel,
        out_type=jax.ShapeDtypeStruct((nb * nsh, 2 * ESIZE), jnp.float32),
        mesh=mesh,
        scratch_types=[
            pltpu.VMEM((xrows_per_w, 2, nsh), jnp.int32),
            pltpu.VMEM((NBUF, nsh, ESIZE), jnp.float32),
            pltpu.VMEM((NBUF, nsh, ESIZE), jnp.float32),
            pltpu.SemaphoreType.DMA((NBUF,)),
            pltpu.SemaphoreType.DMA((NBUF,)),
        ],
        compiler_params=pltpu.CompilerParams(use_tc_tiling_on_sc=False),
    )
    def k(emb_hbm, x_hbm, out_hbm, idx_v, rows_a, rows_b, gsem, ssem):
        wid = lax.axis_index("s") * NC + lax.axis_index("c")
        base = wid * xrows_per_w
        pltpu.sync_copy(x_hbm.at[pl.ds(base, xrows_per_w)], idx_v)

        def gathers(r, b):
            return (
                pltpu.make_async_copy(
                    emb_hbm.at[idx_v.at[r, 0]], rows_a.at[b], gsem.at[b]),
                pltpu.make_async_copy(
                    emb_hbm.at[idx_v.at[r, 1]], rows_b.at[b], gsem.at[b]),
            )

        def stores(r, b):
            p0 = (base + r) * nsh
            return (
                pltpu.make_async_copy(
                    rows_a.at[b], out_hbm.at[pl.ds(p0, nsh), pl.ds(0, ESIZE)],
                    ssem.at[b]),
                pltpu.make_async_copy(
                    rows_b.at[b], out_hbm.at[pl.ds(p0, nsh), pl.ds(ESIZE, ESIZE)],
                    ssem.at[b]),
            )

        def start(descs):
            for d in descs:
                d.start()

        def wait(descs):
            for d in descs:
                d.wait()

        for b in range(NBUF):
            start(gathers(b, b))

        def group(g, _):
            r0 = g * NBUF
            for b in range(NBUF):
                r = r0 + b
                wait(gathers(r, b))
                start(stores(r, b))
                wait(stores(r, b))
                start(gathers(r + NBUF, b))
            return _

        lax.fori_loop(0, ngrp - 1, group, None)

        r0 = (ngrp - 1) * NBUF
        for b in range(NBUF):
            r = r0 + b
            wait(gathers(r, b))
            start(stores(r, b))
            wait(stores(r, b))

    return k(emb, x_eo)


def kernel(x, emb):
    nb, nseq = x.shape
    xi = x.astype(jnp.int32)
    x_eo = jnp.stack([xi[:, 0::2], xi[:, 1::2]], axis=1)
    out = _gather_rows(x_eo, emb)
    return out.reshape(nb, nseq, ESIZE)
